# Initial kernel scaffold; baseline (speedup 1.0000x reference)
import jax
import jax.numpy as jnp
from jax.experimental import pallas as pl


def kernel(x, edge_index, W1, b1, W2, b2):
    raise NotImplementedError("write your pallas kernel here")


# speedup vs baseline: 9.7607x; 9.7607x over previous
import functools

import jax
import jax.numpy as jnp
from jax import lax
from jax.experimental import pallas as pl
from jax.experimental.pallas import tpu as pltpu
from jax.experimental.pallas import tpu_sc as plsc

NC, NS, L = 2, 16, 16
K = 128
KD = 80
G = 32
F32 = jnp.float32


def _deg_fn(NP, EP, GD):
    NW = NC * NS
    CHW = EP // K // NW
    NB = CHW // GD
    RPT = NP // NS
    ZR = 128
    mesh = plsc.VectorSubcoreMesh(core_axis_name="c", subcore_axis_name="s",
                                  num_cores=NC, num_subcores=NS)

    def body(dst5, ones_hbm, zeros_hbm, out, acc, idxv, onesv, zerov):
        cid = lax.axis_index("c")
        sid = lax.axis_index("s")
        w = cid * NS + sid
        pltpu.sync_copy(ones_hbm, onesv)
        pltpu.sync_copy(zeros_hbm, zerov)
        for t in range(RPT // ZR):
            pltpu.sync_copy(zerov, acc.at[pl.ds(sid * RPT + t * ZR, ZR)])
        plsc.subcore_barrier()

        def blk(g, _):
            pltpu.sync_copy(dst5.at[w, g], idxv)

            def lp(i, _):
                pltpu.sync_copy(onesv, acc.at[idxv.at[i, 0]], add=True)
                return 0
            lax.fori_loop(0, GD, lp, 0)
            return 0
        lax.fori_loop(0, NB, blk, 0)

        plsc.subcore_barrier()
        pltpu.sync_copy(acc.at[pl.ds(sid * RPT, RPT)],
                        out.at[cid, pl.ds(sid * RPT, RPT)])

    return pl.kernel(
        body,
        out_type=jax.ShapeDtypeStruct((NC, NP, L), F32),
        mesh=mesh,
        compiler_params=pltpu.CompilerParams(use_tc_tiling_on_sc=False),
        scratch_types=[
            pltpu.VMEM_SHARED((NP, L), F32),
            pltpu.VMEM((GD, 1, K), jnp.int32),
            pltpu.VMEM((K, L), F32),
            pltpu.VMEM((ZR, L), F32),
        ],
    )


def _scatter_fn(NP, EP, D2):
    CH = EP // K // NS
    NB = CH // G
    RPT = NP // NS
    ZR = 128
    mesh = plsc.VectorSubcoreMesh(core_axis_name="c", subcore_axis_name="s", num_cores=NC, num_subcores=NS)

    def body(hpA, hpB, src5, dst5, out, acc, srcv, dstv, rows,
             sem0, sem1):
        cid = lax.axis_index("c")
        sid = lax.axis_index("s")
        zero_row = jnp.zeros((L,), F32)

        def fill(i, _):
            for j in range(D2 // L):
                rows[0, i, pl.ds(j * L, L)] = zero_row
            return 0
        lax.fori_loop(0, ZR, fill, 0)
        for t in range(RPT // ZR):
            pltpu.sync_copy(rows.at[0], acc.at[pl.ds(sid * RPT + t * ZR, ZR)])
        plsc.subcore_barrier()

        def run(table):
            def blk(g, _):
                pltpu.sync_copy(src5.at[sid, g], srcv)
                pltpu.sync_copy(dst5.at[sid, g], dstv)
                pltpu.async_copy(table.at[srcv.at[0, 0]], rows.at[0], sem0)

                def lp(p, _):
                    c0 = 2 * p
                    pltpu.async_copy(table.at[srcv.at[c0 + 1, 0]],
                                     rows.at[1], sem1)
                    pltpu.make_async_copy(table.at[srcv.at[c0, 0]],
                                          rows.at[0], sem0).wait()
                    pltpu.sync_copy(rows.at[0], acc.at[dstv.at[c0, 0]],
                                    add=True)

                    @pl.when(p < G // 2 - 1)
                    def _():
                        pltpu.async_copy(table.at[srcv.at[c0 + 2, 0]],
                                         rows.at[0], sem0)

                    pltpu.make_async_copy(table.at[srcv.at[c0 + 1, 0]],
                                          rows.at[1], sem1).wait()
                    pltpu.sync_copy(rows.at[1], acc.at[dstv.at[c0 + 1, 0]],
                                    add=True)
                    return 0
                lax.fori_loop(0, G // 2, lp, 0)
                return 0
            lax.fori_loop(0, NB, blk, 0)

        @pl.when(cid == 0)
        def _():
            run(hpA)

        @pl.when(cid == 1)
        def _():
            run(hpB)

        plsc.subcore_barrier()
        pltpu.sync_copy(acc.at[pl.ds(sid * RPT, RPT)],
                        out.at[cid, pl.ds(sid * RPT, RPT)])

    return pl.kernel(
        body,
        out_type=jax.ShapeDtypeStruct((NC, NP, D2), F32),
        mesh=mesh,
        scratch_types=[
            pltpu.VMEM_SHARED((NP, D2), F32),
            pltpu.VMEM((G, 1, K), jnp.int32),
            pltpu.VMEM((G, 1, K), jnp.int32),
            pltpu.VMEM((2, K, D2), F32),
            pltpu.SemaphoreType.DMA,
            pltpu.SemaphoreType.DMA,
        ],
    )


def _scatter_edges_fn(NP, EP, D, G2):
    NW = NC * NS
    CHW = EP // K // NW
    NB = CHW // G2
    RPT = NP // NS
    ZR = 128
    mesh = plsc.VectorSubcoreMesh(core_axis_name="c", subcore_axis_name="s", num_cores=NC, num_subcores=NS)

    def body(table, src5, dst5, out, acc, srcv, dstv, rows, sem0, sem1):
        cid = lax.axis_index("c")
        sid = lax.axis_index("s")
        w = cid * NS + sid
        zero_row = jnp.zeros((L,), F32)

        def fill(i, _):
            for j in range(D // L):
                rows[0, i, pl.ds(j * L, L)] = zero_row
            return 0
        lax.fori_loop(0, ZR, fill, 0)
        for t in range(RPT // ZR):
            pltpu.sync_copy(rows.at[0], acc.at[pl.ds(sid * RPT + t * ZR, ZR)])
        plsc.subcore_barrier()

        def blk(g, _):
            pltpu.sync_copy(src5.at[w, g], srcv)
            pltpu.sync_copy(dst5.at[w, g], dstv)
            pltpu.async_copy(table.at[srcv.at[0, 0]], rows.at[0], sem0)

            def lp(p, _):
                c0 = 2 * p
                pltpu.async_copy(table.at[srcv.at[c0 + 1, 0]],
                                 rows.at[1], sem1)
                pltpu.make_async_copy(table.at[srcv.at[c0, 0]],
                                      rows.at[0], sem0).wait()
                pltpu.sync_copy(rows.at[0], acc.at[dstv.at[c0, 0]],
                                add=True)

                @pl.when(p < G2 // 2 - 1)
                def _():
                    pltpu.async_copy(table.at[srcv.at[c0 + 2, 0]],
                                     rows.at[0], sem0)

                pltpu.make_async_copy(table.at[srcv.at[c0 + 1, 0]],
                                      rows.at[1], sem1).wait()
                pltpu.sync_copy(rows.at[1], acc.at[dstv.at[c0 + 1, 0]],
                                add=True)
                return 0
            lax.fori_loop(0, G2 // 2, lp, 0)
            return 0
        lax.fori_loop(0, NB, blk, 0)

        plsc.subcore_barrier()
        pltpu.sync_copy(acc.at[pl.ds(sid * RPT, RPT)],
                        out.at[cid, pl.ds(sid * RPT, RPT)])

    return pl.kernel(
        body,
        out_type=jax.ShapeDtypeStruct((NC, NP, D), F32),
        mesh=mesh,
        scratch_types=[
            pltpu.VMEM_SHARED((NP, D), F32),
            pltpu.VMEM((G2, 1, K), jnp.int32),
            pltpu.VMEM((G2, 1, K), jnp.int32),
            pltpu.VMEM((2, K, D), F32),
            pltpu.SemaphoreType.DMA,
            pltpu.SemaphoreType.DMA,
        ],
    )


BR = 2048


def _row_spec(D):
    return pl.BlockSpec((BR, D), lambda i: (i, 0))


def _full_spec(shape):
    nd = len(shape)
    return pl.BlockSpec(shape, lambda i: (0,) * nd)


def _tc_pre(x, W1, degA, degB):
    NP = x.shape[0]
    din = x.shape[1]
    dh = W1.shape[1]
    D2 = dh // 2

    def body(x_ref, w_ref, da_ref, db_ref, ha_ref, hb_ref, dinv_ref):
        dsum = da_ref[...] + db_ref[...]
        deg = 1.0 + dsum[:, 0:1]
        dinv = lax.rsqrt(deg)
        h = jnp.dot(x_ref[...], w_ref[...],
                    preferred_element_type=F32,
                    precision=lax.Precision.HIGHEST)
        hp = h * dinv
        ha_ref[...] = hp[:, :D2]
        hb_ref[...] = hp[:, D2:]
        dinv_ref[...] = dinv

    return pl.pallas_call(
        body,
        grid=(NP // BR,),
        in_specs=[_row_spec(din), _full_spec(W1.shape),
                  _row_spec(L), _row_spec(L)],
        out_specs=[_row_spec(D2), _row_spec(D2), _row_spec(1)],
        out_shape=[
            jax.ShapeDtypeStruct((NP, D2), F32),
            jax.ShapeDtypeStruct((NP, D2), F32),
            jax.ShapeDtypeStruct((NP, 1), F32),
        ],
    )(x, W1, degA, degB)


def _tc_mid(S1a, S1b, hA, hB, dinv, b1, W2, n_valid):
    NP = S1a.shape[0]
    D2 = S1a.shape[1]
    dout = W2.shape[1]

    def body(sa_ref, sb_ref, ha_ref, hb_ref, dinv_ref, b1_ref, w2_ref,
             o_ref):
        dinv = dinv_ref[...]
        za = jnp.maximum(dinv * (sa_ref[...] + ha_ref[...])
                         + b1_ref[:, :D2], 0.0)
        zb = jnp.maximum(dinv * (sb_ref[...] + hb_ref[...])
                         + b1_ref[:, D2:], 0.0)
        z = jnp.concatenate([za, zb], axis=1)
        h2 = jnp.dot(z, w2_ref[...],
                     preferred_element_type=F32,
                     precision=lax.Precision.HIGHEST)
        row = (pl.program_id(0) * BR
               + lax.broadcasted_iota(jnp.int32, (BR, 1), 0))
        o_ref[...] = jnp.where(row < n_valid, h2 * dinv, 0.0)

    return pl.pallas_call(
        body,
        grid=(NP // BR,),
        in_specs=[_row_spec(D2), _row_spec(D2), _row_spec(D2),
                  _row_spec(D2), _row_spec(1), _full_spec(b1.shape),
                  _full_spec(W2.shape)],
        out_specs=_row_spec(dout),
        out_shape=jax.ShapeDtypeStruct((NP, dout), F32),
    )(S1a, S1b, hA, hB, dinv, b1, W2)


def _tc_post(S2a, S2b, h2p, dinv, b2):
    NP = S2a.shape[0]
    D = S2a.shape[1]

    def body(sa_ref, sb_ref, h_ref, dinv_ref, b2_ref, out_ref):
        dinv = dinv_ref[...]
        out_ref[...] = dinv * (sa_ref[...] + sb_ref[...] + h_ref[...]) \
            + b2_ref[...]

    return pl.pallas_call(
        body,
        grid=(NP // BR,),
        in_specs=[_row_spec(D), _row_spec(D), _row_spec(D),
                  _row_spec(1), _full_spec(b2.shape)],
        out_specs=_row_spec(D),
        out_shape=jax.ShapeDtypeStruct((NP, D), F32),
    )(S2a, S2b, h2p, dinv, b2)


def kernel(x, edge_index, W1, b1, W2, b2):
    N, din = x.shape
    E = edge_index.shape[1]
    dh = W1.shape[1]
    dout = W2.shape[1]
    NP = ((N + BR - 1) // BR) * BR

    xp = jnp.pad(x, ((0, NP - N), (0, 0)))

    blk_edges = NS * G * K
    EP = ((E + blk_edges - 1) // blk_edges) * blk_edges
    pad = EP - E
    src_p = jnp.concatenate(
        [edge_index[0], jnp.full((pad,), N, jnp.int32)])
    dst_p = jnp.concatenate(
        [edge_index[1], (jnp.arange(pad, dtype=jnp.int32) * 61) % N])
    srcS = src_p.reshape(NS, EP // K // NS // G, G, 1, K)
    dstS = dst_p.reshape(NS, EP // K // NS // G, G, 1, K)
    G2 = 16
    srcE = src_p.reshape(NC * NS, EP // K // (NC * NS) // G2, G2, 1, K)
    dstE = dst_p.reshape(NC * NS, EP // K // (NC * NS) // G2, G2, 1, K)

    dst_deg = jnp.concatenate([edge_index[1],
                               jnp.full((pad,), N, jnp.int32)])
    dstD = dst_deg.reshape(NC * NS, EP // K // (NC * NS) // G2, G2, 1, K)
    iota16 = jnp.arange(L, dtype=jnp.int32)
    ones16 = jnp.tile(jnp.where(iota16 == 0, 1.0, 0.0)[None, :],
                      (K, 1)).astype(F32)
    zeros16 = jnp.zeros((128, L), F32)

    degpart = _deg_fn(NP, EP, G2)(dstD, ones16, zeros16)
    hA, hB, dinv = _tc_pre(xp, W1, degpart[0], degpart[1])
    S1 = _scatter_fn(NP, EP, dh // 2)(hA, hB, srcS, dstS)
    h2p = _tc_mid(S1[0], S1[1], hA, hB, dinv,
                  b1.reshape(1, -1), W2, N)
    S2 = _scatter_edges_fn(NP, EP, dout, G2)(h2p, srcE, dstE)
    out = _tc_post(S2[0], S2[1], h2p, dinv, b2.reshape(1, -1))
    return out[:N]


# speedup vs baseline: 9.7658x; 1.0005x over previous
import functools

import jax
import jax.numpy as jnp
from jax import lax
from jax.experimental import pallas as pl
from jax.experimental.pallas import tpu as pltpu
from jax.experimental.pallas import tpu_sc as plsc

NC, NS, L = 2, 16, 16
K = 128
KD = 80
G = 32
F32 = jnp.float32


def _deg_fn(NP, EP, GD):
    NW = NC * NS
    CHW = EP // K // NW
    NB = CHW // GD
    RPT = NP // NS
    ZR = 128
    mesh = plsc.VectorSubcoreMesh(core_axis_name="c", subcore_axis_name="s",
                                  num_cores=NC, num_subcores=NS)

    def body(dst5, ones_hbm, zeros_hbm, out, acc, idxv, onesv, zerov):
        cid = lax.axis_index("c")
        sid = lax.axis_index("s")
        w = cid * NS + sid
        pltpu.sync_copy(ones_hbm, onesv)
        pltpu.sync_copy(zeros_hbm, zerov)
        for t in range(RPT // ZR):
            pltpu.sync_copy(zerov, acc.at[pl.ds(sid * RPT + t * ZR, ZR)])
        plsc.subcore_barrier()

        def blk(g, _):
            pltpu.sync_copy(dst5.at[w, g], idxv)

            def lp(i, _):
                pltpu.sync_copy(onesv, acc.at[idxv.at[i, 0]], add=True)
                return 0
            lax.fori_loop(0, GD, lp, 0)
            return 0
        lax.fori_loop(0, NB, blk, 0)

        plsc.subcore_barrier()
        pltpu.sync_copy(acc.at[pl.ds(sid * RPT, RPT)],
                        out.at[cid, pl.ds(sid * RPT, RPT)])

    return pl.kernel(
        body,
        out_type=jax.ShapeDtypeStruct((NC, NP, L), F32),
        mesh=mesh,
        compiler_params=pltpu.CompilerParams(use_tc_tiling_on_sc=False),
        scratch_types=[
            pltpu.VMEM_SHARED((NP, L), F32),
            pltpu.VMEM((GD, 1, K), jnp.int32),
            pltpu.VMEM((K, L), F32),
            pltpu.VMEM((ZR, L), F32),
        ],
    )


def _scatter_fn(NP, EP, D2):
    CH = EP // K // NS
    NB = CH // G
    RPT = NP // NS
    ZR = 128
    mesh = plsc.VectorSubcoreMesh(core_axis_name="c", subcore_axis_name="s", num_cores=NC, num_subcores=NS)

    def body(hpA, hpB, src5, dst5, out, acc, srcv, dstv, rows,
             sem0, sem1):
        cid = lax.axis_index("c")
        sid = lax.axis_index("s")
        zero_row = jnp.zeros((L,), F32)

        def fill(i, _):
            for j in range(D2 // L):
                rows[0, i, pl.ds(j * L, L)] = zero_row
            return 0
        lax.fori_loop(0, ZR, fill, 0)
        for t in range(RPT // ZR):
            pltpu.sync_copy(rows.at[0], acc.at[pl.ds(sid * RPT + t * ZR, ZR)])
        plsc.subcore_barrier()

        def run(table):
            def blk(g, _):
                pltpu.sync_copy(src5.at[sid, g], srcv)
                pltpu.sync_copy(dst5.at[sid, g], dstv)
                pltpu.async_copy(table.at[srcv.at[0, 0]], rows.at[0], sem0)

                def lp(p, _):
                    c0 = 2 * p
                    pltpu.async_copy(table.at[srcv.at[c0 + 1, 0]],
                                     rows.at[1], sem1)
                    pltpu.make_async_copy(table.at[srcv.at[c0, 0]],
                                          rows.at[0], sem0).wait()
                    pltpu.sync_copy(rows.at[0], acc.at[dstv.at[c0, 0]],
                                    add=True)

                    @pl.when(p < G // 2 - 1)
                    def _():
                        pltpu.async_copy(table.at[srcv.at[c0 + 2, 0]],
                                         rows.at[0], sem0)

                    pltpu.make_async_copy(table.at[srcv.at[c0 + 1, 0]],
                                          rows.at[1], sem1).wait()
                    pltpu.sync_copy(rows.at[1], acc.at[dstv.at[c0 + 1, 0]],
                                    add=True)
                    return 0
                lax.fori_loop(0, G // 2, lp, 0)
                return 0
            lax.fori_loop(0, NB, blk, 0)

        @pl.when(cid == 0)
        def _():
            run(hpA)

        @pl.when(cid == 1)
        def _():
            run(hpB)

        plsc.subcore_barrier()
        pltpu.sync_copy(acc.at[pl.ds(sid * RPT, RPT)],
                        out.at[cid, pl.ds(sid * RPT, RPT)])

    return pl.kernel(
        body,
        out_type=jax.ShapeDtypeStruct((NC, NP, D2), F32),
        mesh=mesh,
        compiler_params=pltpu.CompilerParams(use_tc_tiling_on_sc=False),
        scratch_types=[
            pltpu.VMEM_SHARED((NP, D2), F32),
            pltpu.VMEM((G, 1, K), jnp.int32),
            pltpu.VMEM((G, 1, K), jnp.int32),
            pltpu.VMEM((2, K, D2), F32),
            pltpu.SemaphoreType.DMA,
            pltpu.SemaphoreType.DMA,
        ],
    )


def _scatter_edges_fn(NP, EP, D, G2):
    NW = NC * NS
    CHW = EP // K // NW
    NB = CHW // G2
    RPT = NP // NS
    ZR = 128
    mesh = plsc.VectorSubcoreMesh(core_axis_name="c", subcore_axis_name="s", num_cores=NC, num_subcores=NS)

    def body(table, src5, dst5, out, acc, srcv, dstv, rows, sem0, sem1):
        cid = lax.axis_index("c")
        sid = lax.axis_index("s")
        w = cid * NS + sid
        zero_row = jnp.zeros((L,), F32)

        def fill(i, _):
            for j in range(D // L):
                rows[0, i, pl.ds(j * L, L)] = zero_row
            return 0
        lax.fori_loop(0, ZR, fill, 0)
        for t in range(RPT // ZR):
            pltpu.sync_copy(rows.at[0], acc.at[pl.ds(sid * RPT + t * ZR, ZR)])
        plsc.subcore_barrier()

        def blk(g, _):
            pltpu.sync_copy(src5.at[w, g], srcv)
            pltpu.sync_copy(dst5.at[w, g], dstv)
            pltpu.async_copy(table.at[srcv.at[0, 0]], rows.at[0], sem0)

            def lp(p, _):
                c0 = 2 * p
                pltpu.async_copy(table.at[srcv.at[c0 + 1, 0]],
                                 rows.at[1], sem1)
                pltpu.make_async_copy(table.at[srcv.at[c0, 0]],
                                      rows.at[0], sem0).wait()
                pltpu.sync_copy(rows.at[0], acc.at[dstv.at[c0, 0]],
                                add=True)

                @pl.when(p < G2 // 2 - 1)
                def _():
                    pltpu.async_copy(table.at[srcv.at[c0 + 2, 0]],
                                     rows.at[0], sem0)

                pltpu.make_async_copy(table.at[srcv.at[c0 + 1, 0]],
                                      rows.at[1], sem1).wait()
                pltpu.sync_copy(rows.at[1], acc.at[dstv.at[c0 + 1, 0]],
                                add=True)
                return 0
            lax.fori_loop(0, G2 // 2, lp, 0)
            return 0
        lax.fori_loop(0, NB, blk, 0)

        plsc.subcore_barrier()
        pltpu.sync_copy(acc.at[pl.ds(sid * RPT, RPT)],
                        out.at[cid, pl.ds(sid * RPT, RPT)])

    return pl.kernel(
        body,
        out_type=jax.ShapeDtypeStruct((NC, NP, D), F32),
        mesh=mesh,
        compiler_params=pltpu.CompilerParams(use_tc_tiling_on_sc=False),
        scratch_types=[
            pltpu.VMEM_SHARED((NP, D), F32),
            pltpu.VMEM((G2, 1, K), jnp.int32),
            pltpu.VMEM((G2, 1, K), jnp.int32),
            pltpu.VMEM((2, K, D), F32),
            pltpu.SemaphoreType.DMA,
            pltpu.SemaphoreType.DMA,
        ],
    )


BR = 2048


def _row_spec(D):
    return pl.BlockSpec((BR, D), lambda i: (i, 0))


def _full_spec(shape):
    nd = len(shape)
    return pl.BlockSpec(shape, lambda i: (0,) * nd)


def _tc_pre(x, W1, degA, degB):
    NP = x.shape[0]
    din = x.shape[1]
    dh = W1.shape[1]
    D2 = dh // 2

    def body(x_ref, w_ref, da_ref, db_ref, ha_ref, hb_ref, dinv_ref):
        dsum = da_ref[...] + db_ref[...]
        deg = 1.0 + dsum[:, 0:1]
        dinv = lax.rsqrt(deg)
        h = jnp.dot(x_ref[...], w_ref[...],
                    preferred_element_type=F32,
                    precision=lax.Precision.HIGHEST)
        hp = h * dinv
        ha_ref[...] = hp[:, :D2]
        hb_ref[...] = hp[:, D2:]
        dinv_ref[...] = dinv

    return pl.pallas_call(
        body,
        grid=(NP // BR,),
        in_specs=[_row_spec(din), _full_spec(W1.shape),
                  _row_spec(L), _row_spec(L)],
        out_specs=[_row_spec(D2), _row_spec(D2), _row_spec(1)],
        out_shape=[
            jax.ShapeDtypeStruct((NP, D2), F32),
            jax.ShapeDtypeStruct((NP, D2), F32),
            jax.ShapeDtypeStruct((NP, 1), F32),
        ],
    )(x, W1, degA, degB)


def _tc_mid(S1a, S1b, hA, hB, dinv, b1, W2, n_valid):
    NP = S1a.shape[0]
    D2 = S1a.shape[1]
    dout = W2.shape[1]

    def body(sa_ref, sb_ref, ha_ref, hb_ref, dinv_ref, b1_ref, w2_ref,
             o_ref):
        dinv = dinv_ref[...]
        za = jnp.maximum(dinv * (sa_ref[...] + ha_ref[...])
                         + b1_ref[:, :D2], 0.0)
        zb = jnp.maximum(dinv * (sb_ref[...] + hb_ref[...])
                         + b1_ref[:, D2:], 0.0)
        z = jnp.concatenate([za, zb], axis=1)
        h2 = jnp.dot(z, w2_ref[...],
                     preferred_element_type=F32,
                     precision=lax.Precision.HIGHEST)
        row = (pl.program_id(0) * BR
               + lax.broadcasted_iota(jnp.int32, (BR, 1), 0))
        o_ref[...] = jnp.where(row < n_valid, h2 * dinv, 0.0)

    return pl.pallas_call(
        body,
        grid=(NP // BR,),
        in_specs=[_row_spec(D2), _row_spec(D2), _row_spec(D2),
                  _row_spec(D2), _row_spec(1), _full_spec(b1.shape),
                  _full_spec(W2.shape)],
        out_specs=_row_spec(dout),
        out_shape=jax.ShapeDtypeStruct((NP, dout), F32),
    )(S1a, S1b, hA, hB, dinv, b1, W2)


def _tc_post(S2a, S2b, h2p, dinv, b2):
    NP = S2a.shape[0]
    D = S2a.shape[1]

    def body(sa_ref, sb_ref, h_ref, dinv_ref, b2_ref, out_ref):
        dinv = dinv_ref[...]
        out_ref[...] = dinv * (sa_ref[...] + sb_ref[...] + h_ref[...]) \
            + b2_ref[...]

    return pl.pallas_call(
        body,
        grid=(NP // BR,),
        in_specs=[_row_spec(D), _row_spec(D), _row_spec(D),
                  _row_spec(1), _full_spec(b2.shape)],
        out_specs=_row_spec(D),
        out_shape=jax.ShapeDtypeStruct((NP, D), F32),
    )(S2a, S2b, h2p, dinv, b2)


def kernel(x, edge_index, W1, b1, W2, b2):
    N, din = x.shape
    E = edge_index.shape[1]
    dh = W1.shape[1]
    dout = W2.shape[1]
    NP = ((N + BR - 1) // BR) * BR

    xp = jnp.pad(x, ((0, NP - N), (0, 0)))

    blk_edges = NS * G * K
    EP = ((E + blk_edges - 1) // blk_edges) * blk_edges
    pad = EP - E
    src_p = jnp.concatenate(
        [edge_index[0], jnp.full((pad,), N, jnp.int32)])
    dst_p = jnp.concatenate(
        [edge_index[1], (jnp.arange(pad, dtype=jnp.int32) * 61) % N])
    srcS = src_p.reshape(NS, EP // K // NS // G, G, 1, K)
    dstS = dst_p.reshape(NS, EP // K // NS // G, G, 1, K)
    G2 = 16
    srcE = src_p.reshape(NC * NS, EP // K // (NC * NS) // G2, G2, 1, K)
    dstE = dst_p.reshape(NC * NS, EP // K // (NC * NS) // G2, G2, 1, K)

    dst_deg = jnp.concatenate([edge_index[1],
                               jnp.full((pad,), N, jnp.int32)])
    dstD = dst_deg.reshape(NC * NS, EP // K // (NC * NS) // G2, G2, 1, K)
    iota16 = jnp.arange(L, dtype=jnp.int32)
    ones16 = jnp.tile(jnp.where(iota16 == 0, 1.0, 0.0)[None, :],
                      (K, 1)).astype(F32)
    zeros16 = jnp.zeros((128, L), F32)

    degpart = _deg_fn(NP, EP, G2)(dstD, ones16, zeros16)
    hA, hB, dinv = _tc_pre(xp, W1, degpart[0], degpart[1])
    S1 = _scatter_fn(NP, EP, dh // 2)(hA, hB, srcS, dstS)
    h2p = _tc_mid(S1[0], S1[1], hA, hB, dinv,
                  b1.reshape(1, -1), W2, N)
    S2 = _scatter_edges_fn(NP, EP, dout, G2)(h2p, srcE, dstE)
    out = _tc_post(S2[0], S2[1], h2p, dinv, b2.reshape(1, -1))
    return out[:N]
